# baseline (speedup 1.0000x reference)
import jax
import jax.numpy as jnp
from jax import lax
from jax.experimental import pallas as pl
from jax.experimental.pallas import tpu as pltpu
from jax.experimental.pallas import tpu_sc as plsc

N = 100000
E = 1600000
NODE_DIM = 47
HIDDEN = 128
LATENT = 64
NUM_GRAPHS = 64

NC, NS, L = 2, 16, 16

ECOLS = 128
EROWS = 12512
NPIECES = EROWS // 16
EPAD = EROWS * ECOLS - E

N_A = 100352
R_CHUNK = 10240
C_CHUNKS = -(-N // R_CHUNK)
SHIFT = 17

ROW_BLK = 2048
N_BLOCKS = (N + ROW_BLK - 1) // ROW_BLK

_SC_MESH = plsc.VectorSubcoreMesh(
    core_axis_name="c", subcore_axis_name="s", num_cores=NC, num_subcores=NS)


def _deg_body(dst_hbm, hist2_hbm, acc_sp, dbuf, e0buf, zbuf, sem):
    c = lax.axis_index("c")
    s = lax.axis_index("s")
    wid = c * NS + s

    zeros16 = jnp.zeros((L,), jnp.float32)
    e0 = jnp.where(jnp.arange(L, dtype=jnp.int32) == 0, 1.0, 0.0)
    for r in range(16):
        zbuf[r, :] = zeros16
        e0buf[r, :] = e0

    def _z(i, _):
        pltpu.sync_copy(zbuf, acc_sp.at[pl.ds(s * (N_A // NS) + i * 16, 16)])
        return 0
    lax.fori_loop(0, N_A // NS // 16, _z, 0)
    plsc.subcore_barrier()

    npw = (NPIECES - wid + NC * NS - 1) // (NC * NS)

    def _piece(i, _):
        p = wid + i * (NC * NS)
        pltpu.sync_copy(dst_hbm.at[pl.ds(p * 16, 16)], dbuf)

        def _row(j, _):
            descs = []
            for k in range(ECOLS // L):
                didx = dbuf[j, pl.ds(k * L, L)]
                descs.append(
                    pltpu.async_copy(e0buf, acc_sp.at[didx], sem, add=True))
            for d in descs:
                d.wait()
            return 0
        lax.fori_loop(0, 16, _row, 0)
        return 0
    lax.fori_loop(0, npw, _piece, 0)
    plsc.subcore_barrier()

    pltpu.sync_copy(acc_sp.at[pl.ds(s * (N_A // NS), N_A // NS)],
                    hist2_hbm.at[c, pl.ds(s * (N_A // NS), N_A // NS)])


def _deg_hist(dstp):
    return pl.kernel(
        _deg_body,
        out_type=jax.ShapeDtypeStruct((NC, N_A, L), jnp.float32),
        mesh=_SC_MESH,
        compiler_params=pltpu.CompilerParams(use_tc_tiling_on_sc=False),
        scratch_types=[
            pltpu.VMEM_SHARED((N_A, L), jnp.float32),
            pltpu.VMEM((16, ECOLS), jnp.int32),
            pltpu.VMEM((16, L), jnp.float32),
            pltpu.VMEM((16, L), jnp.float32),
            pltpu.SemaphoreType.DMA,
        ],
    )(dstp)


N_ACC = 100096


def _make_scatter_body(P):
    def body(gflat_hbm, src_hbm, dst_hbm, out_hbm,
             acc_sp, sbuf, dbuf, sgbuf, grbuf, zbuf, sem_g, sem_s):
        c = lax.axis_index("c")
        s = lax.axis_index("s")

        zeros16 = jnp.zeros((L,), jnp.float32)
        for r in range(8):
            zbuf[r, :] = zeros16

        def _pass(pi, _):
            p = c * (P // 2) + pi

            def _z(i, _):
                pltpu.sync_copy(
                    zbuf, acc_sp.at[pl.ds(s * (N_ACC // NS) + i * 8, 8)])
                return 0
            lax.fori_loop(0, N_ACC // NS // 8, _z, 0)
            plsc.subcore_barrier()

            npw = (NPIECES - s + NS - 1) // NS

            def _prep_fire(t):
                q3 = t % 3
                row = t // 2
                base = (t % 2) * 64
                for u in range(4):
                    sgbuf[q3, pl.ds(u * L, L)] = (
                        sbuf[row, pl.ds(base + u * L, L)] * P + p)
                pltpu.async_copy(gflat_hbm.at[sgbuf.at[q3]], grbuf.at[q3],
                                 sem_g)

            def _scatter4(t):
                q3 = t % 3
                row = t // 2
                base = (t % 2) * 64
                for u in range(4):
                    sgbuf[3 + q3, pl.ds(u * L, L)] = dbuf[row, pl.ds(
                        base + u * L, L)]
                pltpu.async_copy(grbuf.at[q3], acc_sp.at[sgbuf.at[3 + q3]],
                                 sem_s, add=True)

            def _drain_s(n):
                for _u in range(n):
                    pltpu.make_async_copy(grbuf.at[0],
                                          acc_sp.at[pl.ds(0, 64)], sem_s).wait()

            def _wait_g():
                pltpu.make_async_copy(gflat_hbm.at[pl.ds(0, 64)],
                                      grbuf.at[0], sem_g).wait()

            def _piece(i, _):
                pc = s + i * NS
                pltpu.sync_copy(src_hbm.at[pl.ds(pc * 16, 16)], sbuf)
                pltpu.sync_copy(dst_hbm.at[pl.ds(pc * 16, 16)], dbuf)

                _prep_fire(jnp.int32(0))
                _prep_fire(jnp.int32(1))

                def _trio(jj, _):
                    for q in range(3):
                        t = 3 * jj + q
                        if q == 0:
                            @pl.when(jj > 0)
                            def _():
                                _drain_s(1)
                        else:
                            _drain_s(1)
                        _prep_fire(t + 2)
                        _wait_g()
                        _scatter4(t)
                    return 0
                lax.fori_loop(0, 10, _trio, 0)
                for tt in (30, 31):
                    _wait_g()
                    _scatter4(jnp.int32(tt))
                _drain_s(3)
                return 0
            lax.fori_loop(0, npw, _piece, 0)
            plsc.subcore_barrier()

            pltpu.sync_copy(
                acc_sp.at[pl.ds(s * (N_ACC // NS), N_ACC // NS)],
                out_hbm.at[p, pl.ds(s * (N_ACC // NS), N_ACC // NS)])
            pltpu.sync_copy(zbuf, out_hbm.at[p, pl.ds(N_ACC + s * 16, 8)])
            pltpu.sync_copy(zbuf, out_hbm.at[p, pl.ds(N_ACC + s * 16 + 8, 8)])
            plsc.subcore_barrier()
            return 0
        lax.fori_loop(0, P // 2, _pass, 0)
    return body


def _edge_scatter(gflat, srcp, dstp, P):
    body = _make_scatter_body(P)
    return pl.kernel(
        body,
        out_type=jax.ShapeDtypeStruct((P, N_A, L), jnp.float32),
        mesh=_SC_MESH,
        compiler_params=pltpu.CompilerParams(use_tc_tiling_on_sc=False),
        scratch_types=[
            pltpu.VMEM_SHARED((N_ACC, L), jnp.float32),
            pltpu.VMEM((16, ECOLS), jnp.int32),
            pltpu.VMEM((16, ECOLS), jnp.int32),
            pltpu.VMEM((6, 64), jnp.int32),
            pltpu.VMEM((3, 64, L), jnp.float32),
            pltpu.VMEM((8, L), jnp.float32),
            pltpu.SemaphoreType.DMA,
            pltpu.SemaphoreType.DMA,
        ],
    )(gflat, srcp, dstp)


def _embed_body(x_ref, we_ref, be_ref, w1_ref, hist_ref, g1_ref, dinv_ref):
    deg = 1.0 + hist_ref[0, :, 0:1] + hist_ref[1, :, 0:1]
    dinv = lax.rsqrt(deg)
    h0 = jnp.maximum(x_ref[...] @ we_ref[...] + be_ref[...][None, :], 0.0)
    g1_ref[...] = (h0 @ w1_ref[...]) * dinv
    dinv_ref[...] = dinv


def _embed_matmul(x, W_emb, b_emb, W1, hist2):
    return pl.pallas_call(
        _embed_body,
        grid=(N_BLOCKS,),
        in_specs=[
            pl.BlockSpec((ROW_BLK, NODE_DIM), lambda i: (i, 0)),
            pl.BlockSpec((NODE_DIM, HIDDEN), lambda i: (0, 0)),
            pl.BlockSpec((HIDDEN,), lambda i: (0,)),
            pl.BlockSpec((HIDDEN, HIDDEN), lambda i: (0, 0)),
            pl.BlockSpec((NC, ROW_BLK, L), lambda i: (0, i, 0)),
        ],
        out_specs=[
            pl.BlockSpec((ROW_BLK, HIDDEN), lambda i: (i, 0)),
            pl.BlockSpec((ROW_BLK, 1), lambda i: (i, 0)),
        ],
        out_shape=[
            jax.ShapeDtypeStruct((N, HIDDEN), jnp.float32),
            jax.ShapeDtypeStruct((N, 1), jnp.float32),
        ],
    )(x, W_emb, b_emb, W1, hist2)


def _combine_body(acc_ref, g_ref, dinv_ref, b_ref, w2_ref, out_ref):
    dinv = dinv_ref[...]
    acc = jnp.concatenate(
        [acc_ref[q] for q in range(HIDDEN // L)], axis=-1)
    h1 = jnp.maximum(dinv * (acc + g_ref[...]) + b_ref[...][None, :], 0.0)
    out_ref[...] = (h1 @ w2_ref[...]) * dinv


def _combine_matmul(acc1, g1, dinv, b1, W2):
    return pl.pallas_call(
        _combine_body,
        grid=(N_BLOCKS,),
        in_specs=[
            pl.BlockSpec((HIDDEN // L, ROW_BLK, L), lambda i: (0, i, 0)),
            pl.BlockSpec((ROW_BLK, HIDDEN), lambda i: (i, 0)),
            pl.BlockSpec((ROW_BLK, 1), lambda i: (i, 0)),
            pl.BlockSpec((HIDDEN,), lambda i: (0,)),
            pl.BlockSpec((HIDDEN, LATENT), lambda i: (0, 0)),
        ],
        out_specs=pl.BlockSpec((ROW_BLK, LATENT), lambda i: (i, 0)),
        out_shape=jax.ShapeDtypeStruct((N, LATENT), jnp.float32),
    )(acc1, g1, dinv, b1, W2)


def _pool_body(acc_ref, g_ref, dinv_ref, b_ref, batch_ref, out_ref,
               sum_ref, cnt_ref):
    i = pl.program_id(0)

    @pl.when(i == 0)
    def _init():
        sum_ref[...] = jnp.zeros_like(sum_ref)
        cnt_ref[...] = jnp.zeros_like(cnt_ref)

    acc = jnp.concatenate(
        [acc_ref[q] for q in range(LATENT // L)], axis=-1)
    h2 = jnp.maximum(
        dinv_ref[...] * (acc + g_ref[...]) + b_ref[...][None, :], 0.0)
    bb = batch_ref[...]
    rows = i * ROW_BLK + lax.broadcasted_iota(jnp.int32, (ROW_BLK, 1), 0)
    valid = rows < N
    onehot = jnp.where(
        (bb[:, None] == lax.broadcasted_iota(jnp.int32, (1, NUM_GRAPHS), 1))
        & valid, 1.0, 0.0)
    sum_ref[...] += jax.lax.dot_general(
        onehot, h2, (((0,), (0,)), ((), ())),
        preferred_element_type=jnp.float32)
    cnt_ref[...] += jnp.sum(onehot, axis=0)[:, None]

    @pl.when(i == pl.num_programs(0) - 1)
    def _fin():
        out_ref[...] = sum_ref[...] / jnp.maximum(cnt_ref[...], 1.0)


def _combine_pool(acc2, g2, dinv, b2, batch):
    return pl.pallas_call(
        _pool_body,
        grid=(N_BLOCKS,),
        in_specs=[
            pl.BlockSpec((LATENT // L, ROW_BLK, L), lambda i: (0, i, 0)),
            pl.BlockSpec((ROW_BLK, LATENT), lambda i: (i, 0)),
            pl.BlockSpec((ROW_BLK, 1), lambda i: (i, 0)),
            pl.BlockSpec((LATENT,), lambda i: (0,)),
            pl.BlockSpec((ROW_BLK,), lambda i: (i,)),
        ],
        out_specs=pl.BlockSpec((NUM_GRAPHS, LATENT), lambda i: (0, 0)),
        out_shape=jax.ShapeDtypeStruct((NUM_GRAPHS, LATENT), jnp.float32),
        scratch_shapes=[
            pltpu.VMEM((NUM_GRAPHS, LATENT), jnp.float32),
            pltpu.VMEM((NUM_GRAPHS, 1), jnp.float32),
        ],
    )(acc2, g2, dinv, b2, batch)


def kernel(x, edge_index, batch, W_emb, b_emb, W1, b1, W2, b2):
    src, dst = edge_index[0], edge_index[1]
    srcp = jnp.concatenate(
        [src, jnp.zeros((EPAD,), jnp.int32)]).reshape(EROWS, ECOLS)
    dstp = jnp.concatenate(
        [dst, jnp.full((EPAD,), N, jnp.int32)]).reshape(EROWS, ECOLS)

    hist2 = _deg_hist(dstp)
    g1, dinv = _embed_matmul(x, W_emb, b_emb, W1, hist2)
    acc1 = _edge_scatter(g1.reshape(N * (HIDDEN // L), L), srcp, dstp,
                         HIDDEN // L)
    g2 = _combine_matmul(acc1, g1, dinv, b1, W2)
    acc2 = _edge_scatter(g2.reshape(N * (LATENT // L), L), srcp, dstp,
                         LATENT // L)
    return _combine_pool(acc2, g2, dinv, b2, batch)


# speedup vs baseline: 1.0595x; 1.0595x over previous
import jax
import jax.numpy as jnp
from jax import lax
from jax.experimental import pallas as pl
from jax.experimental.pallas import tpu as pltpu
from jax.experimental.pallas import tpu_sc as plsc

N = 100000
E = 1600000
NODE_DIM = 47
HIDDEN = 128
LATENT = 64
NUM_GRAPHS = 64

NC, NS, L = 2, 16, 16

ECOLS = 128
EROWS = 12512
NPIECES = EROWS // 16
EPAD = EROWS * ECOLS - E

N_A = 100352
R_CHUNK = 10240
C_CHUNKS = -(-N // R_CHUNK)
SHIFT = 17

ROW_BLK = 2048
N_BLOCKS = (N + ROW_BLK - 1) // ROW_BLK

_SC_MESH = plsc.VectorSubcoreMesh(
    core_axis_name="c", subcore_axis_name="s", num_cores=NC, num_subcores=NS)


def _deg_body(dst_hbm, hist2_hbm, acc_sp, dbuf, e0buf, zbuf, sem):
    c = lax.axis_index("c")
    s = lax.axis_index("s")
    wid = c * NS + s

    zeros16 = jnp.zeros((L,), jnp.float32)
    e0 = jnp.where(jnp.arange(L, dtype=jnp.int32) == 0, 1.0, 0.0)
    for r in range(16):
        zbuf[r, :] = zeros16
        e0buf[r, :] = e0

    def _z(i, _):
        pltpu.sync_copy(zbuf, acc_sp.at[pl.ds(s * (N_A // NS) + i * 16, 16)])
        return 0
    lax.fori_loop(0, N_A // NS // 16, _z, 0)
    plsc.subcore_barrier()

    npw = (NPIECES - wid + NC * NS - 1) // (NC * NS)

    def _piece(i, _):
        p = wid + i * (NC * NS)
        pltpu.sync_copy(dst_hbm.at[pl.ds(p * 16, 16)], dbuf)

        def _row(j, _):
            descs = []
            for k in range(ECOLS // L):
                didx = dbuf[j, pl.ds(k * L, L)]
                descs.append(
                    pltpu.async_copy(e0buf, acc_sp.at[didx], sem, add=True))
            for d in descs:
                d.wait()
            return 0
        lax.fori_loop(0, 16, _row, 0)
        return 0
    lax.fori_loop(0, npw, _piece, 0)
    plsc.subcore_barrier()

    pltpu.sync_copy(acc_sp.at[pl.ds(s * (N_A // NS), N_A // NS)],
                    hist2_hbm.at[c, pl.ds(s * (N_A // NS), N_A // NS)])


def _deg_hist(dstp):
    return pl.kernel(
        _deg_body,
        out_type=jax.ShapeDtypeStruct((NC, N_A, L), jnp.float32),
        mesh=_SC_MESH,
        compiler_params=pltpu.CompilerParams(use_tc_tiling_on_sc=False),
        scratch_types=[
            pltpu.VMEM_SHARED((N_A, L), jnp.float32),
            pltpu.VMEM((16, ECOLS), jnp.int32),
            pltpu.VMEM((16, L), jnp.float32),
            pltpu.VMEM((16, L), jnp.float32),
            pltpu.SemaphoreType.DMA,
        ],
    )(dstp)


N_ACC = 100096


def _make_scatter_body(P):
    def body(gflat_hbm, src_hbm, dst_hbm, out_hbm,
             acc_sp, sbuf, dbuf, sgbuf, grbuf, zbuf, sem_g, sem_s):
        c = lax.axis_index("c")
        s = lax.axis_index("s")

        zeros16 = jnp.zeros((L,), jnp.float32)
        for r in range(8):
            zbuf[r, :] = zeros16

        def _pass(pi, _):
            p = c * (P // 2) + pi

            def _z(i, _):
                pltpu.sync_copy(
                    zbuf, acc_sp.at[pl.ds(s * (N_ACC // NS) + i * 8, 8)])
                return 0
            lax.fori_loop(0, N_ACC // NS // 8, _z, 0)
            plsc.subcore_barrier()

            npw = (NPIECES - s + NS - 1) // NS

            def _prep_fire(t):
                q3 = t % 3
                row = t // 2
                base = (t % 2) * 64
                for u in range(4):
                    sgbuf[q3, pl.ds(u * L, L)] = (
                        sbuf[row, pl.ds(base + u * L, L)] * P + p)
                pltpu.async_copy(gflat_hbm.at[sgbuf.at[q3]], grbuf.at[q3],
                                 sem_g)

            def _scatter4(t):
                q3 = t % 3
                row = t // 2
                base = (t % 2) * 64
                for u in range(4):
                    didx16 = dbuf[row, pl.ds(base + u * L, L)]
                    pltpu.async_copy(grbuf.at[q3, pl.ds(u * L, L)],
                                     acc_sp.at[didx16], sem_s, add=True)

            def _drain_s(n):
                for _u in range(n):
                    pltpu.make_async_copy(grbuf.at[0, pl.ds(0, L)],
                                          acc_sp.at[pl.ds(0, L)], sem_s).wait()

            def _wait_g():
                pltpu.make_async_copy(gflat_hbm.at[pl.ds(0, 64)],
                                      grbuf.at[0], sem_g).wait()

            def _piece(i, _):
                pc = s + i * NS
                d1 = pltpu.async_copy(src_hbm.at[pl.ds(pc * 16, 16)], sbuf,
                                      sem_g)
                d2 = pltpu.async_copy(dst_hbm.at[pl.ds(pc * 16, 16)], dbuf,
                                      sem_g)
                d1.wait()
                d2.wait()

                _prep_fire(jnp.int32(0))
                _prep_fire(jnp.int32(1))

                def _trio(jj, _):
                    for q in range(3):
                        t = 3 * jj + q
                        if q == 0:
                            @pl.when(jj > 0)
                            def _():
                                _drain_s(4)
                        else:
                            _drain_s(4)
                        _prep_fire(t + 2)
                        _wait_g()
                        _scatter4(t)
                    return 0
                lax.fori_loop(0, 10, _trio, 0)
                for tt in (30, 31):
                    _wait_g()
                    _scatter4(jnp.int32(tt))
                _drain_s(12)
                return 0
            lax.fori_loop(0, npw, _piece, 0)
            plsc.subcore_barrier()

            pltpu.sync_copy(
                acc_sp.at[pl.ds(s * (N_ACC // NS), N_ACC // NS)],
                out_hbm.at[p, pl.ds(s * (N_ACC // NS), N_ACC // NS)])
            pltpu.sync_copy(zbuf, out_hbm.at[p, pl.ds(N_ACC + s * 16, 8)])
            pltpu.sync_copy(zbuf, out_hbm.at[p, pl.ds(N_ACC + s * 16 + 8, 8)])
            plsc.subcore_barrier()
            return 0
        lax.fori_loop(0, P // 2, _pass, 0)
    return body


def _edge_scatter(gflat, srcp, dstp, P):
    body = _make_scatter_body(P)
    return pl.kernel(
        body,
        out_type=jax.ShapeDtypeStruct((P, N_A, L), jnp.float32),
        mesh=_SC_MESH,
        compiler_params=pltpu.CompilerParams(use_tc_tiling_on_sc=False),
        scratch_types=[
            pltpu.VMEM_SHARED((N_ACC, L), jnp.float32),
            pltpu.VMEM((16, ECOLS), jnp.int32),
            pltpu.VMEM((16, ECOLS), jnp.int32),
            pltpu.VMEM((3, 64), jnp.int32),
            pltpu.VMEM((3, 64, L), jnp.float32),
            pltpu.VMEM((8, L), jnp.float32),
            pltpu.SemaphoreType.DMA,
            pltpu.SemaphoreType.DMA,
        ],
    )(gflat, srcp, dstp)


def _embed_body(x_ref, we_ref, be_ref, w1_ref, hist_ref, g1_ref, dinv_ref):
    deg = 1.0 + hist_ref[0, :, 0:1] + hist_ref[1, :, 0:1]
    dinv = lax.rsqrt(deg)
    h0 = jnp.maximum(x_ref[...] @ we_ref[...] + be_ref[...][None, :], 0.0)
    g1_ref[...] = (h0 @ w1_ref[...]) * dinv
    dinv_ref[...] = dinv


def _embed_matmul(x, W_emb, b_emb, W1, hist2):
    return pl.pallas_call(
        _embed_body,
        grid=(N_BLOCKS,),
        in_specs=[
            pl.BlockSpec((ROW_BLK, NODE_DIM), lambda i: (i, 0)),
            pl.BlockSpec((NODE_DIM, HIDDEN), lambda i: (0, 0)),
            pl.BlockSpec((HIDDEN,), lambda i: (0,)),
            pl.BlockSpec((HIDDEN, HIDDEN), lambda i: (0, 0)),
            pl.BlockSpec((NC, ROW_BLK, L), lambda i: (0, i, 0)),
        ],
        out_specs=[
            pl.BlockSpec((ROW_BLK, HIDDEN), lambda i: (i, 0)),
            pl.BlockSpec((ROW_BLK, 1), lambda i: (i, 0)),
        ],
        out_shape=[
            jax.ShapeDtypeStruct((N, HIDDEN), jnp.float32),
            jax.ShapeDtypeStruct((N, 1), jnp.float32),
        ],
    )(x, W_emb, b_emb, W1, hist2)


def _combine_body(acc_ref, g_ref, dinv_ref, b_ref, w2_ref, out_ref):
    dinv = dinv_ref[...]
    acc = jnp.concatenate(
        [acc_ref[q] for q in range(HIDDEN // L)], axis=-1)
    h1 = jnp.maximum(dinv * (acc + g_ref[...]) + b_ref[...][None, :], 0.0)
    out_ref[...] = (h1 @ w2_ref[...]) * dinv


def _combine_matmul(acc1, g1, dinv, b1, W2):
    return pl.pallas_call(
        _combine_body,
        grid=(N_BLOCKS,),
        in_specs=[
            pl.BlockSpec((HIDDEN // L, ROW_BLK, L), lambda i: (0, i, 0)),
            pl.BlockSpec((ROW_BLK, HIDDEN), lambda i: (i, 0)),
            pl.BlockSpec((ROW_BLK, 1), lambda i: (i, 0)),
            pl.BlockSpec((HIDDEN,), lambda i: (0,)),
            pl.BlockSpec((HIDDEN, LATENT), lambda i: (0, 0)),
        ],
        out_specs=pl.BlockSpec((ROW_BLK, LATENT), lambda i: (i, 0)),
        out_shape=jax.ShapeDtypeStruct((N, LATENT), jnp.float32),
    )(acc1, g1, dinv, b1, W2)


def _pool_body(acc_ref, g_ref, dinv_ref, b_ref, batch_ref, out_ref,
               sum_ref, cnt_ref):
    i = pl.program_id(0)

    @pl.when(i == 0)
    def _init():
        sum_ref[...] = jnp.zeros_like(sum_ref)
        cnt_ref[...] = jnp.zeros_like(cnt_ref)

    acc = jnp.concatenate(
        [acc_ref[q] for q in range(LATENT // L)], axis=-1)
    h2 = jnp.maximum(
        dinv_ref[...] * (acc + g_ref[...]) + b_ref[...][None, :], 0.0)
    bb = batch_ref[...]
    rows = i * ROW_BLK + lax.broadcasted_iota(jnp.int32, (ROW_BLK, 1), 0)
    valid = rows < N
    onehot = jnp.where(
        (bb[:, None] == lax.broadcasted_iota(jnp.int32, (1, NUM_GRAPHS), 1))
        & valid, 1.0, 0.0)
    sum_ref[...] += jax.lax.dot_general(
        onehot, h2, (((0,), (0,)), ((), ())),
        preferred_element_type=jnp.float32)
    cnt_ref[...] += jnp.sum(onehot, axis=0)[:, None]

    @pl.when(i == pl.num_programs(0) - 1)
    def _fin():
        out_ref[...] = sum_ref[...] / jnp.maximum(cnt_ref[...], 1.0)


def _combine_pool(acc2, g2, dinv, b2, batch):
    return pl.pallas_call(
        _pool_body,
        grid=(N_BLOCKS,),
        in_specs=[
            pl.BlockSpec((LATENT // L, ROW_BLK, L), lambda i: (0, i, 0)),
            pl.BlockSpec((ROW_BLK, LATENT), lambda i: (i, 0)),
            pl.BlockSpec((ROW_BLK, 1), lambda i: (i, 0)),
            pl.BlockSpec((LATENT,), lambda i: (0,)),
            pl.BlockSpec((ROW_BLK,), lambda i: (i,)),
        ],
        out_specs=pl.BlockSpec((NUM_GRAPHS, LATENT), lambda i: (0, 0)),
        out_shape=jax.ShapeDtypeStruct((NUM_GRAPHS, LATENT), jnp.float32),
        scratch_shapes=[
            pltpu.VMEM((NUM_GRAPHS, LATENT), jnp.float32),
            pltpu.VMEM((NUM_GRAPHS, 1), jnp.float32),
        ],
    )(acc2, g2, dinv, b2, batch)


def kernel(x, edge_index, batch, W_emb, b_emb, W1, b1, W2, b2):
    src, dst = edge_index[0], edge_index[1]
    srcp = jnp.concatenate(
        [src, jnp.zeros((EPAD,), jnp.int32)]).reshape(EROWS, ECOLS)
    dstp = jnp.concatenate(
        [dst, jnp.full((EPAD,), N, jnp.int32)]).reshape(EROWS, ECOLS)

    hist2 = _deg_hist(dstp)
    g1, dinv = _embed_matmul(x, W_emb, b_emb, W1, hist2)
    acc1 = _edge_scatter(g1.reshape(N * (HIDDEN // L), L), srcp, dstp,
                         HIDDEN // L)
    g2 = _combine_matmul(acc1, g1, dinv, b1, W2)
    acc2 = _edge_scatter(g2.reshape(N * (LATENT // L), L), srcp, dstp,
                         LATENT // L)
    return _combine_pool(acc2, g2, dinv, b2, batch)


# speedup vs baseline: 1.1354x; 1.0717x over previous
import jax
import jax.numpy as jnp
from jax import lax
from jax.experimental import pallas as pl
from jax.experimental.pallas import tpu as pltpu
from jax.experimental.pallas import tpu_sc as plsc

N = 100000
E = 1600000
NODE_DIM = 47
HIDDEN = 128
LATENT = 64
NUM_GRAPHS = 64

NC, NS, L = 2, 16, 16

ECOLS = 128
EROWS = 12512
NPIECES = EROWS // 16
EPAD = EROWS * ECOLS - E

N_A = 100352
R_CHUNK = 10240
C_CHUNKS = -(-N // R_CHUNK)
SHIFT = 17

ROW_BLK = 2048
N_BLOCKS = (N + ROW_BLK - 1) // ROW_BLK

_SC_MESH = plsc.VectorSubcoreMesh(
    core_axis_name="c", subcore_axis_name="s", num_cores=NC, num_subcores=NS)


def _deg_body(dst_hbm, hist2_hbm, acc_sp, dbuf, e0buf, zbuf, sem):
    c = lax.axis_index("c")
    s = lax.axis_index("s")
    wid = c * NS + s

    zeros16 = jnp.zeros((L,), jnp.float32)
    e0 = jnp.where(jnp.arange(L, dtype=jnp.int32) == 0, 1.0, 0.0)
    for r in range(16):
        zbuf[r, :] = zeros16
        e0buf[r, :] = e0

    def _z(i, _):
        pltpu.sync_copy(zbuf, acc_sp.at[pl.ds(s * (N_A // NS) + i * 16, 16)])
        return 0
    lax.fori_loop(0, N_A // NS // 16, _z, 0)
    plsc.subcore_barrier()

    npw = (NPIECES - wid + NC * NS - 1) // (NC * NS)

    def _piece(i, _):
        p = wid + i * (NC * NS)
        pltpu.sync_copy(dst_hbm.at[pl.ds(p * 16, 16)], dbuf)

        def _row(j, _):
            descs = []
            for k in range(ECOLS // L):
                didx = dbuf[j, pl.ds(k * L, L)]
                descs.append(
                    pltpu.async_copy(e0buf, acc_sp.at[didx], sem, add=True))
            for d in descs:
                d.wait()
            return 0
        lax.fori_loop(0, 16, _row, 0)
        return 0
    lax.fori_loop(0, npw, _piece, 0)
    plsc.subcore_barrier()

    pltpu.sync_copy(acc_sp.at[pl.ds(s * (N_A // NS), N_A // NS)],
                    hist2_hbm.at[c, pl.ds(s * (N_A // NS), N_A // NS)])


def _deg_hist(dstp):
    return pl.kernel(
        _deg_body,
        out_type=jax.ShapeDtypeStruct((NC, N_A, L), jnp.float32),
        mesh=_SC_MESH,
        compiler_params=pltpu.CompilerParams(use_tc_tiling_on_sc=False),
        scratch_types=[
            pltpu.VMEM_SHARED((N_A, L), jnp.float32),
            pltpu.VMEM((16, ECOLS), jnp.int32),
            pltpu.VMEM((16, L), jnp.float32),
            pltpu.VMEM((16, L), jnp.float32),
            pltpu.SemaphoreType.DMA,
        ],
    )(dstp)


N_ACC = 100096


def _make_scatter_body(P):
    def body(gflat_hbm, src_hbm, dst_hbm, out_hbm,
             acc_sp, sbuf, dbuf, sgbuf, grbuf, zbuf, sem_g, sem_s):
        c = lax.axis_index("c")
        s = lax.axis_index("s")

        zeros16 = jnp.zeros((L,), jnp.float32)
        for r in range(8):
            zbuf[r, :] = zeros16

        def _pass(pi, _):
            p = c * (P // 2) + pi

            def _z(i, _):
                pltpu.sync_copy(
                    zbuf, acc_sp.at[pl.ds(s * (N_ACC // NS) + i * 8, 8)])
                return 0
            lax.fori_loop(0, N_ACC // NS // 8, _z, 0)
            plsc.subcore_barrier()

            npw = (NPIECES - s + NS - 1) // NS

            def _prep_fire(t):
                q3 = t % 3
                row = t // 2
                base = (t % 2) * 64
                for u in range(4):
                    sgbuf[q3, pl.ds(u * L, L)] = (
                        sbuf[row, pl.ds(base + u * L, L)] * P + p)
                pltpu.async_copy(gflat_hbm.at[sgbuf.at[q3]], grbuf.at[q3],
                                 sem_g)

            def _scatter4(t):
                q3 = t % 3
                row = t // 2
                base = (t % 2) * 64
                for u in range(4):
                    didx16 = dbuf[row, pl.ds(base + u * L, L)]
                    pltpu.async_copy(grbuf.at[q3, pl.ds(u * L, L)],
                                     acc_sp.at[didx16], sem_s, add=True)

            def _drain_s(n):
                for _u in range(n):
                    pltpu.make_async_copy(grbuf.at[0, pl.ds(0, L)],
                                          acc_sp.at[pl.ds(0, L)], sem_s).wait()

            def _wait_g():
                pltpu.make_async_copy(gflat_hbm.at[pl.ds(0, 64)],
                                      grbuf.at[0], sem_g).wait()

            def _piece(i, _):
                pc = s + i * NS
                d1 = pltpu.async_copy(src_hbm.at[pl.ds(pc * 16, 16)], sbuf,
                                      sem_g)
                d2 = pltpu.async_copy(dst_hbm.at[pl.ds(pc * 16, 16)], dbuf,
                                      sem_g)
                d1.wait()
                d2.wait()

                _prep_fire(jnp.int32(0))
                _prep_fire(jnp.int32(1))

                def _trio(jj, _):
                    for q in range(3):
                        t = 3 * jj + q
                        if q == 0:
                            @pl.when(jj > 0)
                            def _():
                                _drain_s(4)
                        else:
                            _drain_s(4)
                        _prep_fire(t + 2)
                        _wait_g()
                        _scatter4(t)
                    return 0
                lax.fori_loop(0, 10, _trio, 0)
                for tt in (30, 31):
                    _wait_g()
                    _scatter4(jnp.int32(tt))
                _drain_s(12)
                return 0
            lax.fori_loop(0, npw, _piece, 0)
            plsc.subcore_barrier()

            pltpu.sync_copy(
                acc_sp.at[pl.ds(s * (N_ACC // NS), N_ACC // NS)],
                out_hbm.at[pl.ds(s * (N_ACC // NS), N_ACC // NS),
                           pl.ds(p * L, L)])
            pltpu.sync_copy(zbuf, out_hbm.at[pl.ds(N_ACC + s * 16, 8),
                                             pl.ds(p * L, L)])
            pltpu.sync_copy(zbuf, out_hbm.at[pl.ds(N_ACC + s * 16 + 8, 8),
                                             pl.ds(p * L, L)])
            plsc.subcore_barrier()
            return 0
        lax.fori_loop(0, P // 2, _pass, 0)
    return body


def _edge_scatter(gflat, srcp, dstp, P):
    body = _make_scatter_body(P)
    return pl.kernel(
        body,
        out_type=jax.ShapeDtypeStruct((N_A, P * L), jnp.float32),
        mesh=_SC_MESH,
        compiler_params=pltpu.CompilerParams(use_tc_tiling_on_sc=False),
        scratch_types=[
            pltpu.VMEM_SHARED((N_ACC, L), jnp.float32),
            pltpu.VMEM((16, ECOLS), jnp.int32),
            pltpu.VMEM((16, ECOLS), jnp.int32),
            pltpu.VMEM((3, 64), jnp.int32),
            pltpu.VMEM((3, 64, L), jnp.float32),
            pltpu.VMEM((8, L), jnp.float32),
            pltpu.SemaphoreType.DMA,
            pltpu.SemaphoreType.DMA,
        ],
    )(gflat, srcp, dstp)


def _embed_body(x_ref, we_ref, be_ref, w1_ref, hist_ref, g1_ref, dinv_ref):
    deg = 1.0 + hist_ref[0, :, 0:1] + hist_ref[1, :, 0:1]
    dinv = lax.rsqrt(deg)
    h0 = jnp.maximum(x_ref[...] @ we_ref[...] + be_ref[...][None, :], 0.0)
    g1_ref[...] = (h0 @ w1_ref[...]) * dinv
    dinv_ref[...] = dinv


def _embed_matmul(x, W_emb, b_emb, W1, hist2):
    return pl.pallas_call(
        _embed_body,
        grid=(N_BLOCKS,),
        in_specs=[
            pl.BlockSpec((ROW_BLK, NODE_DIM), lambda i: (i, 0)),
            pl.BlockSpec((NODE_DIM, HIDDEN), lambda i: (0, 0)),
            pl.BlockSpec((HIDDEN,), lambda i: (0,)),
            pl.BlockSpec((HIDDEN, HIDDEN), lambda i: (0, 0)),
            pl.BlockSpec((NC, ROW_BLK, L), lambda i: (0, i, 0)),
        ],
        out_specs=[
            pl.BlockSpec((ROW_BLK, HIDDEN), lambda i: (i, 0)),
            pl.BlockSpec((ROW_BLK, 1), lambda i: (i, 0)),
        ],
        out_shape=[
            jax.ShapeDtypeStruct((N, HIDDEN), jnp.float32),
            jax.ShapeDtypeStruct((N, 1), jnp.float32),
        ],
    )(x, W_emb, b_emb, W1, hist2)


def _combine_body(acc_ref, g_ref, dinv_ref, b_ref, w2_ref, out_ref):
    dinv = dinv_ref[...]
    h1 = jnp.maximum(dinv * (acc_ref[...] + g_ref[...]) + b_ref[...][None, :],
                     0.0)
    out_ref[...] = (h1 @ w2_ref[...]) * dinv


def _combine_matmul(acc1, g1, dinv, b1, W2):
    return pl.pallas_call(
        _combine_body,
        grid=(N_BLOCKS,),
        in_specs=[
            pl.BlockSpec((ROW_BLK, HIDDEN), lambda i: (i, 0)),
            pl.BlockSpec((ROW_BLK, HIDDEN), lambda i: (i, 0)),
            pl.BlockSpec((ROW_BLK, 1), lambda i: (i, 0)),
            pl.BlockSpec((HIDDEN,), lambda i: (0,)),
            pl.BlockSpec((HIDDEN, LATENT), lambda i: (0, 0)),
        ],
        out_specs=pl.BlockSpec((ROW_BLK, LATENT), lambda i: (i, 0)),
        out_shape=jax.ShapeDtypeStruct((N, LATENT), jnp.float32),
    )(acc1, g1, dinv, b1, W2)


def _pool_body(acc_ref, g_ref, dinv_ref, b_ref, batch_ref, out_ref,
               sum_ref, cnt_ref):
    i = pl.program_id(0)

    @pl.when(i == 0)
    def _init():
        sum_ref[...] = jnp.zeros_like(sum_ref)
        cnt_ref[...] = jnp.zeros_like(cnt_ref)

    h2 = jnp.maximum(
        dinv_ref[...] * (acc_ref[...] + g_ref[...]) + b_ref[...][None, :],
        0.0)
    bb = batch_ref[...]
    rows = i * ROW_BLK + lax.broadcasted_iota(jnp.int32, (ROW_BLK, 1), 0)
    valid = rows < N
    onehot = jnp.where(
        (bb[:, None] == lax.broadcasted_iota(jnp.int32, (1, NUM_GRAPHS), 1))
        & valid, 1.0, 0.0)
    sum_ref[...] += jax.lax.dot_general(
        onehot, h2, (((0,), (0,)), ((), ())),
        preferred_element_type=jnp.float32)
    cnt_ref[...] += jnp.sum(onehot, axis=0)[:, None]

    @pl.when(i == pl.num_programs(0) - 1)
    def _fin():
        out_ref[...] = sum_ref[...] / jnp.maximum(cnt_ref[...], 1.0)


def _combine_pool(acc2, g2, dinv, b2, batch):
    return pl.pallas_call(
        _pool_body,
        grid=(N_BLOCKS,),
        in_specs=[
            pl.BlockSpec((ROW_BLK, LATENT), lambda i: (i, 0)),
            pl.BlockSpec((ROW_BLK, LATENT), lambda i: (i, 0)),
            pl.BlockSpec((ROW_BLK, 1), lambda i: (i, 0)),
            pl.BlockSpec((LATENT,), lambda i: (0,)),
            pl.BlockSpec((ROW_BLK,), lambda i: (i,)),
        ],
        out_specs=pl.BlockSpec((NUM_GRAPHS, LATENT), lambda i: (0, 0)),
        out_shape=jax.ShapeDtypeStruct((NUM_GRAPHS, LATENT), jnp.float32),
        scratch_shapes=[
            pltpu.VMEM((NUM_GRAPHS, LATENT), jnp.float32),
            pltpu.VMEM((NUM_GRAPHS, 1), jnp.float32),
        ],
    )(acc2, g2, dinv, b2, batch)


def kernel(x, edge_index, batch, W_emb, b_emb, W1, b1, W2, b2):
    src, dst = edge_index[0], edge_index[1]
    srcp = jnp.concatenate(
        [src, jnp.zeros((EPAD,), jnp.int32)]).reshape(EROWS, ECOLS)
    dstp = jnp.concatenate(
        [dst, jnp.full((EPAD,), N, jnp.int32)]).reshape(EROWS, ECOLS)

    hist2 = _deg_hist(dstp)
    g1, dinv = _embed_matmul(x, W_emb, b_emb, W1, hist2)
    acc1 = _edge_scatter(g1.reshape(N * (HIDDEN // L), L), srcp, dstp,
                         HIDDEN // L)
    g2 = _combine_matmul(acc1, g1, dinv, b1, W2)
    acc2 = _edge_scatter(g2.reshape(N * (LATENT // L), L), srcp, dstp,
                         LATENT // L)
    return _combine_pool(acc2, g2, dinv, b2, batch)
